# baseline (speedup 1.0000x reference)
import jax
import jax.numpy as jnp
from jax.experimental import pallas as pl

_NUM_CLASSES = 1000
_MULT = 26
_ROWS = 1024
_BLOCK_BATCH = 128


def _onehot_block(feat_ref, out_ref):
    ft = feat_ref[...].T
    classes = jax.lax.broadcasted_iota(
        jnp.int32, (_MULT, _NUM_CLASSES, _BLOCK_BATCH), 1
    )
    out_ref[...] = (ft[:, None, :] == classes).astype(jnp.int32)


def kernel(feat):
    grid = (_ROWS // _BLOCK_BATCH,)
    t = pl.pallas_call(
        _onehot_block,
        grid=grid,
        in_specs=[pl.BlockSpec((_BLOCK_BATCH, _MULT), lambda i: (i, 0))],
        out_specs=pl.BlockSpec(
            (_MULT, _NUM_CLASSES, _BLOCK_BATCH), lambda i: (0, 0, i)
        ),
        out_shape=jax.ShapeDtypeStruct((_MULT, _NUM_CLASSES, _ROWS), jnp.int32),
    )(feat)
    return jnp.transpose(t, (2, 0, 1))


# speedup vs baseline: 1.0272x; 1.0272x over previous
import jax
import jax.numpy as jnp
from jax.experimental import pallas as pl

_NUM_CLASSES = 1000
_MULT = 26
_ROWS = 1024
_BLOCK_CLS = 200


def _onehot_block(featT_ref, out_ref):
    i = pl.program_id(0)
    f = featT_ref[...]
    classes = jax.lax.broadcasted_iota(
        jnp.int32, (_MULT, _BLOCK_CLS, _ROWS), 1
    ) + i * _BLOCK_CLS
    out_ref[...] = (f[:, None, :] == classes).astype(jnp.int32)


def kernel(feat):
    featT = feat.T
    grid = (_NUM_CLASSES // _BLOCK_CLS,)
    t = pl.pallas_call(
        _onehot_block,
        grid=grid,
        in_specs=[pl.BlockSpec((_MULT, _ROWS), lambda i: (0, 0))],
        out_specs=pl.BlockSpec(
            (_MULT, _BLOCK_CLS, _ROWS), lambda i: (0, i, 0)
        ),
        out_shape=jax.ShapeDtypeStruct((_MULT, _NUM_CLASSES, _ROWS), jnp.int32),
    )(featT)
    return jnp.transpose(t, (2, 0, 1))


# speedup vs baseline: 1.0703x; 1.0420x over previous
import jax
import jax.numpy as jnp
from jax.experimental import pallas as pl

_NUM_CLASSES = 1000
_MULT = 26
_ROWS = 1024
_BLOCK_BATCH = 128


def _onehot_block(featT_ref, out_ref):
    f = featT_ref[...]
    classes = jax.lax.broadcasted_iota(
        jnp.int32, (_MULT, _NUM_CLASSES, _BLOCK_BATCH), 1
    )
    out_ref[...] = (f[:, None, :] == classes).astype(jnp.int32)


def kernel(feat):
    featT = feat.T
    grid = (_ROWS // _BLOCK_BATCH,)
    t = pl.pallas_call(
        _onehot_block,
        grid=grid,
        in_specs=[pl.BlockSpec((_MULT, _BLOCK_BATCH), lambda i: (0, i))],
        out_specs=pl.BlockSpec(
            (_MULT, _NUM_CLASSES, _BLOCK_BATCH), lambda i: (0, 0, i)
        ),
        out_shape=jax.ShapeDtypeStruct((_MULT, _NUM_CLASSES, _ROWS), jnp.int32),
    )(featT)
    return jnp.transpose(t, (2, 0, 1))
